# baseline (speedup 1.0000x reference)
import functools

import numpy as np
import jax
import jax.numpy as jnp
from jax import lax
from jax.experimental import pallas as pl
from jax.experimental.pallas import tpu as pltpu
from jax.experimental.pallas import tpu_sc as plsc

N = 10000
E = 320000
NHID = 128
NGRAPH = 64
NOUT = 64

NC, NS = 2, 16
NW = NC * NS
EPT = E // NW
CHUNK = 40
NCHUNK = EPT // CHUNK
NBUF = 5
KITER = NCHUNK // NBUF
RPS = 624
LAST_EXTRA = N - NS * RPS
ZR = 16
NROW = 1000
NBLK = N // NROW
EROW = 2000
EBLK = E // EROW

_PERM = np.array([32 * (p // 32) + (p % 32) // 2 + (16 if p % 2 else 0)
                  for p in range(NHID)], dtype=np.int32)


def _edge_body(h_hbm, e_hbm, src_hbm, dst_hbm, out_hbm,
               agg_sh, sbuf_b, dbuf_b, ebuf_b, rows_b, zed_v,
               sem_i, sem_d, sem_g, sem_s):
    c = lax.axis_index("c")
    s = lax.axis_index("s")
    wid = c * NS + s

    z = jnp.zeros((16,), jnp.float32)

    def zb(i, carry):
        zed_v[i // 8, pl.ds((i % 8) * 16, 16)] = z
        return carry

    lax.fori_loop(0, ZR * 8, zb, 0)
    for k in range(RPS // ZR):
        pltpu.sync_copy(zed_v, agg_sh.at[pl.ds(s * RPS + k * ZR, ZR)])

    @pl.when(s == NS - 1)
    def _():
        pltpu.sync_copy(zed_v, agg_sh.at[pl.ds(NS * RPS, LAST_EXTRA)])

    plsc.subcore_barrier()

    base0 = wid * EPT

    def _src(t, b):
        return pltpu.make_async_copy(
            src_hbm.at[pl.ds(base0 + t * CHUNK, CHUNK)], sbuf_b[b], sem_i[b])

    def _attr(t, b):
        return pltpu.make_async_copy(
            e_hbm.at[pl.ds((base0 + t * CHUNK) * (NHID // 2),
                           CHUNK * (NHID // 2))],
            ebuf_b[b], sem_i[b])

    def _dst(t, b):
        return pltpu.make_async_copy(
            dst_hbm.at[pl.ds(base0 + t * CHUNK, CHUNK)], dbuf_b[b], sem_d[b])

    def _gather(t, b):
        return pltpu.make_async_copy(
            h_hbm.at[sbuf_b[b]], rows_b[b], sem_g[b])

    def _scatter(t, b):
        return pltpu.make_async_copy(
            rows_b[b], agg_sh.at[dbuf_b[b]], sem_s[b])

    def _compute(t, b):
        rows_v = rows_b[b]
        ebuf = ebuf_b[b]

        def edge_body(i, icarry):
            for g in range(4):
                iv = ebuf[pl.ds(i * (NHID // 2) + g * 16, 16)]
                ea = lax.bitcast_convert_type(iv << 16, jnp.float32)
                eb = lax.bitcast_convert_type(iv & jnp.int32(-65536),
                                              jnp.float32)
                m0 = jnp.maximum(rows_v[i, pl.ds(g * 32, 16)] + ea, 0.0)
                rows_v[i, pl.ds(g * 32, 16)] = m0
                m1 = jnp.maximum(rows_v[i, pl.ds(g * 32 + 16, 16)] + eb, 0.0)
                rows_v[i, pl.ds(g * 32 + 16, 16)] = m1
            return icarry

        lax.fori_loop(0, CHUNK, edge_body, 0)

    for b in range(NBUF):
        _src(b, b).start()
        _attr(b, b).start()
        _dst(b, b).start()
    for b in range(NBUF):
        _src(b, b).wait()
        _attr(b, b).wait()
        _gather(b, b).start()

    def kiter(k, carry):
        for b in range(NBUF):
            t = k * NBUF + b
            _gather(t, b).wait()
            _dst(t, b).wait()
            _compute(t, b)
            _scatter(t, b).start(add=True)

            @pl.when(t + NBUF < NCHUNK)
            def _():
                _src(t + NBUF, b).start()
                _attr(t + NBUF, b).start()

        @pl.when(k + 1 < KITER)
        def _():
            for b in range(NBUF):
                tn = (k + 1) * NBUF + b
                _scatter(tn - NBUF, b).wait()
                _dst(tn, b).start()
                _src(tn, b).wait()
                _attr(tn, b).wait()
                _gather(tn, b).start()
        return carry

    lax.fori_loop(0, KITER, kiter, 0)
    for b in range(NBUF):
        _scatter((KITER - 1) * NBUF + b, b).wait()

    plsc.subcore_barrier()
    pltpu.sync_copy(agg_sh.at[pl.ds(s * RPS, RPS)],
                    out_hbm.at[c, pl.ds(s * RPS, RPS)])

    @pl.when(s == NS - 1)
    def _():
        pltpu.sync_copy(agg_sh.at[pl.ds(NS * RPS, LAST_EXTRA)],
                        out_hbm.at[c, pl.ds(NS * RPS, LAST_EXTRA)])


def _edge_pass(h, e_flat, src, dst):
    mesh = plsc.VectorSubcoreMesh(core_axis_name="c", subcore_axis_name="s",
                                  num_cores=NC, num_subcores=NS)
    f = pl.kernel(
        _edge_body,
        out_type=jax.ShapeDtypeStruct((NC, N, NHID), jnp.float32),
        mesh=mesh,
        scratch_types=[
            pltpu.VMEM_SHARED((N, NHID), jnp.float32),
            [pltpu.VMEM((CHUNK,), jnp.int32) for _ in range(NBUF)],
            [pltpu.VMEM((CHUNK,), jnp.int32) for _ in range(NBUF)],
            [pltpu.VMEM((CHUNK * NHID // 2,), jnp.int32) for _ in range(NBUF)],
            [pltpu.VMEM((CHUNK, NHID), jnp.float32) for _ in range(NBUF)],
            pltpu.VMEM((ZR, NHID), jnp.float32),
            [pltpu.SemaphoreType.DMA for _ in range(NBUF)],
            [pltpu.SemaphoreType.DMA for _ in range(NBUF)],
            [pltpu.SemaphoreType.DMA for _ in range(NBUF)],
            [pltpu.SemaphoreType.DMA for _ in range(NBUF)],
        ],
    )
    return f(h, e_flat, src, dst)


def _e_encode_body(a_ref, w_ref, b_ref, o_ref):
    v = jnp.dot(a_ref[...], w_ref[...], preferred_element_type=jnp.float32)
    o_ref[...] = jnp.maximum(v + b_ref[...], 0.0).astype(jnp.bfloat16)


def _e_encode(edge_attr, w, b):
    return pl.pallas_call(
        _e_encode_body,
        grid=(EBLK,),
        in_specs=[pl.BlockSpec((EROW, 4), lambda i: (i, 0)),
                  pl.BlockSpec((4, NHID), lambda i: (0, 0)),
                  pl.BlockSpec((1, NHID), lambda i: (0, 0))],
        out_specs=pl.BlockSpec((EROW, NHID), lambda i: (i, 0)),
        out_shape=jax.ShapeDtypeStruct((E, NHID), jnp.bfloat16),
    )(edge_attr, w, b.reshape(1, NHID))


def _encode_body(x_ref, w_ref, b_ref, o_ref):
    v = jnp.dot(x_ref[...], w_ref[...], preferred_element_type=jnp.float32)
    o_ref[...] = jnp.maximum(v + b_ref[...], 0.0)


def _encode(x, w, b):
    return pl.pallas_call(
        _encode_body,
        grid=(NBLK,),
        in_specs=[pl.BlockSpec((NROW, NHID), lambda i: (i, 0)),
                  pl.BlockSpec((NHID, NHID), lambda i: (0, 0)),
                  pl.BlockSpec((1, NHID), lambda i: (0, 0))],
        out_specs=pl.BlockSpec((NROW, NHID), lambda i: (i, 0)),
        out_shape=jax.ShapeDtypeStruct((N, NHID), jnp.float32),
    )(x, w, b.reshape(1, NHID))


def _combine_body(h_ref, a_ref, w_ref, b_ref, o_ref):
    u = h_ref[...] + a_ref[0] + a_ref[1]
    v = jnp.dot(u, w_ref[...], preferred_element_type=jnp.float32) + b_ref[...]
    o_ref[...] = jnp.maximum(v, 0.0) + h_ref[...]


def _combine(h, agg, w, b):
    return pl.pallas_call(
        _combine_body,
        grid=(NBLK,),
        in_specs=[pl.BlockSpec((NROW, NHID), lambda i: (i, 0)),
                  pl.BlockSpec((NC, NROW, NHID), lambda i: (0, i, 0)),
                  pl.BlockSpec((NHID, NHID), lambda i: (0, 0)),
                  pl.BlockSpec((1, NHID), lambda i: (0, 0))],
        out_specs=pl.BlockSpec((NROW, NHID), lambda i: (i, 0)),
        out_shape=jax.ShapeDtypeStruct((N, NHID), jnp.float32),
    )(h, agg, w, b.reshape(1, NHID))


def _pool_body(h_ref, bt_ref, w1_ref, b1_ref, w2_ref, b2_ref, o_ref, acc_ref):
    i = pl.program_id(0)

    @pl.when(i == 0)
    def _():
        acc_ref[...] = jnp.zeros_like(acc_ref)

    row = bt_ref[0]
    oh = (lax.broadcasted_iota(jnp.int32, (NGRAPH, NROW), 0) == row)
    acc_ref[...] += jnp.dot(oh.astype(jnp.float32), h_ref[...],
                            preferred_element_type=jnp.float32)

    @pl.when(i == NBLK - 1)
    def _():
        p = jnp.dot(acc_ref[...], w1_ref[...],
                    preferred_element_type=jnp.float32) + b1_ref[...]
        p = jnp.maximum(p, 0.0)
        o_ref[...] = jnp.dot(p, w2_ref[...],
                             preferred_element_type=jnp.float32) + b2_ref[...]


def _pool(h, batch3, w1, b1, w2, b2):
    return pl.pallas_call(
        _pool_body,
        grid=(NBLK,),
        in_specs=[pl.BlockSpec((NROW, NHID), lambda i: (i, 0)),
                  pl.BlockSpec((1, 1, NROW), lambda i: (i, 0, 0)),
                  pl.BlockSpec((NHID, NHID), lambda i: (0, 0)),
                  pl.BlockSpec((1, NHID), lambda i: (0, 0)),
                  pl.BlockSpec((NHID, NOUT), lambda i: (0, 0)),
                  pl.BlockSpec((1, NOUT), lambda i: (0, 0))],
        out_specs=pl.BlockSpec((NGRAPH, NOUT), lambda i: (0, 0)),
        out_shape=jax.ShapeDtypeStruct((NGRAPH, NOUT), jnp.float32),
        scratch_shapes=[pltpu.VMEM((NGRAPH, NHID), jnp.float32)],
    )(h, batch3, w1, b1.reshape(1, NHID), w2, b2.reshape(1, NOUT))


def kernel(x, edge_attr, W_in, b_in, W_e, b_e, W_c, b_c, W_o1, b_o1, W_o2,
           b_o2, edge_index, batch):
    src = edge_index[0]
    dst = edge_index[1]
    perm = jnp.asarray(_PERM)
    h = _encode(x, W_in, b_in)
    for l in range(3):
        e2 = _e_encode(edge_attr, W_e[l][:, perm], b_e[l][perm])
        e_i32 = jax.lax.bitcast_convert_type(
            e2.reshape(E * NHID // 2, 2), jnp.int32)
        agg = _edge_pass(h, e_i32, src, dst)
        h = _combine(h, agg, W_c[l], b_c[l])
    return _pool(h, batch.reshape(NBLK, 1, NROW), W_o1, b_o1, W_o2, b_o2)


# speedup vs baseline: 26.6081x; 26.6081x over previous
import functools

import numpy as np
import jax
import jax.numpy as jnp
from jax import lax
from jax.experimental import pallas as pl
from jax.experimental.pallas import tpu as pltpu
from jax.experimental.pallas import tpu_sc as plsc

N = 10000
E = 320000
NHID = 128
NGRAPH = 64
NOUT = 64

NC, NS = 2, 16
NW = NC * NS
EPT = E // NW
CHUNK = 40
NCHUNK = EPT // CHUNK
NBUF = 5
KITER = NCHUNK // NBUF
RPS = 624
LAST_EXTRA = N - NS * RPS
ZR = 16
NROW = 1000
NBLK = N // NROW
EROW = 1000
EBLK = E // 2 // EROW


def _edge_body(h_hbm, e_hbm, src_hbm, dst_hbm, out_hbm,
               agg_sh, sbuf_b, dbuf_b, ebuf_b, rows_b, zed_v,
               sem_i, sem_d, sem_g, sem_s):
    c = lax.axis_index("c")
    s = lax.axis_index("s")
    wid = c * NS + s

    z = jnp.zeros((16,), jnp.float32)

    def zb(i, carry):
        zed_v[i // 8, pl.ds((i % 8) * 16, 16)] = z
        return carry

    lax.fori_loop(0, ZR * 8, zb, 0)
    for k in range(RPS // ZR):
        pltpu.sync_copy(zed_v, agg_sh.at[pl.ds(s * RPS + k * ZR, ZR)])

    @pl.when(s == NS - 1)
    def _():
        pltpu.sync_copy(zed_v, agg_sh.at[pl.ds(NS * RPS, LAST_EXTRA)])

    plsc.subcore_barrier()

    base0 = wid * EPT

    def _src(t, b):
        return pltpu.make_async_copy(
            src_hbm.at[pl.ds(base0 + t * CHUNK, CHUNK)], sbuf_b[b], sem_i[b])

    def _attr(t, b):
        return pltpu.make_async_copy(
            e_hbm.at[pl.ds((base0 + t * CHUNK) * (NHID // 2),
                           CHUNK * (NHID // 2))],
            ebuf_b[b], sem_i[b])

    def _dst(t, b):
        return pltpu.make_async_copy(
            dst_hbm.at[pl.ds(base0 + t * CHUNK, CHUNK)], dbuf_b[b], sem_d[b])

    def _gather(t, b):
        return pltpu.make_async_copy(
            h_hbm.at[sbuf_b[b]], rows_b[b], sem_g[b])

    def _scatter(t, b):
        return pltpu.make_async_copy(
            rows_b[b], agg_sh.at[dbuf_b[b]], sem_s[b])

    def _compute(t, b):
        rows_v = rows_b[b]
        ebuf = ebuf_b[b]

        def edge_body(i, icarry):
            for g in range(4):
                iv = ebuf[pl.ds(i * (NHID // 2) + g * 16, 16)]
                ea = lax.bitcast_convert_type(iv << 16, jnp.float32)
                eb = lax.bitcast_convert_type(iv & jnp.int32(-65536),
                                              jnp.float32)
                m0 = jnp.maximum(rows_v[i, pl.ds(g * 16, 16)] + ea, 0.0)
                rows_v[i, pl.ds(g * 16, 16)] = m0
                m1 = jnp.maximum(rows_v[i, pl.ds(64 + g * 16, 16)] + eb, 0.0)
                rows_v[i, pl.ds(64 + g * 16, 16)] = m1
            return icarry

        lax.fori_loop(0, CHUNK, edge_body, 0)

    LOOK = 3

    for t in range(NBUF):
        _src(t, t).start()
        _attr(t, t).start()
    _dst(0, 0).start()
    for t in range(LOOK):
        _src(t, t).wait()
        _attr(t, t).wait()
        _gather(t, t).start()

    def kiter(k, carry):
        for b in range(NBUF):
            t = k * NBUF + b
            _gather(t, b).wait()
            _dst(t, b).wait()
            _compute(t, b)
            _scatter(t, b).start(add=True)

            @pl.when(t + NBUF < NCHUNK)
            def _():
                _src(t + NBUF, b).start()
                _attr(t + NBUF, b).start()

            @pl.when(t + 1 < NCHUNK)
            def _():
                _dst(t + 1, (b + 1) % NBUF).start()

            @pl.when(t + LOOK < NCHUNK)
            def _():
                bl = (b + LOOK) % NBUF

                @pl.when(t >= NBUF - LOOK)
                def _():
                    _scatter(t - (NBUF - LOOK), bl).wait()

                _src(t + LOOK, bl).wait()
                _attr(t + LOOK, bl).wait()
                _gather(t + LOOK, bl).start()
        return carry

    lax.fori_loop(0, KITER, kiter, 0)
    for t in range(NCHUNK - NBUF, NCHUNK):
        _scatter(t, t % NBUF).wait()

    plsc.subcore_barrier()
    pltpu.sync_copy(agg_sh.at[pl.ds(s * RPS, RPS)],
                    out_hbm.at[c, pl.ds(s * RPS, RPS)])

    @pl.when(s == NS - 1)
    def _():
        pltpu.sync_copy(agg_sh.at[pl.ds(NS * RPS, LAST_EXTRA)],
                        out_hbm.at[c, pl.ds(NS * RPS, LAST_EXTRA)])


def _edge_pass(h, e_flat, src, dst):
    mesh = plsc.VectorSubcoreMesh(core_axis_name="c", subcore_axis_name="s",
                                  num_cores=NC, num_subcores=NS)
    f = pl.kernel(
        _edge_body,
        out_type=jax.ShapeDtypeStruct((NC, N, NHID), jnp.float32),
        mesh=mesh,
        scratch_types=[
            pltpu.VMEM_SHARED((N, NHID), jnp.float32),
            [pltpu.VMEM((CHUNK,), jnp.int32) for _ in range(NBUF)],
            [pltpu.VMEM((CHUNK,), jnp.int32) for _ in range(NBUF)],
            [pltpu.VMEM((CHUNK * NHID // 2,), jnp.int32) for _ in range(NBUF)],
            [pltpu.VMEM((CHUNK, NHID), jnp.float32) for _ in range(NBUF)],
            pltpu.VMEM((ZR, NHID), jnp.float32),
            [pltpu.SemaphoreType.DMA for _ in range(NBUF)],
            [pltpu.SemaphoreType.DMA for _ in range(NBUF)],
            [pltpu.SemaphoreType.DMA for _ in range(NBUF)],
            [pltpu.SemaphoreType.DMA for _ in range(NBUF)],
        ],
    )
    return f(h, e_flat, src, dst)


def _e_encode_body(a_ref, w_ref, b_ref, o_ref):
    halves = []
    for q in range(2):
        a = a_ref[:, 4 * q:4 * q + 4]
        v = jnp.dot(a, w_ref[...], preferred_element_type=jnp.float32)
        v = jnp.maximum(v + b_ref[...], 0.0)
        bits = lax.bitcast_convert_type(v, jnp.int32)
        rnd = (bits + 0x7FFF + ((bits >> 16) & 1)) >> 16
        halves.append((rnd[:, :64] & 0xFFFF) | (rnd[:, 64:] << 16))
    o_ref[...] = jnp.concatenate(halves, axis=1)


def _e_encode(edge_attr2, w, b):
    return pl.pallas_call(
        _e_encode_body,
        grid=(EBLK,),
        in_specs=[pl.BlockSpec((EROW, 8), lambda i: (i, 0)),
                  pl.BlockSpec((4, NHID), lambda i: (0, 0)),
                  pl.BlockSpec((1, NHID), lambda i: (0, 0))],
        out_specs=pl.BlockSpec((EROW, NHID), lambda i: (i, 0)),
        out_shape=jax.ShapeDtypeStruct((E // 2, NHID), jnp.int32),
    )(edge_attr2, w, b.reshape(1, NHID))


def _encode_body(x_ref, w_ref, b_ref, o_ref):
    v = jnp.dot(x_ref[...], w_ref[...], preferred_element_type=jnp.float32)
    o_ref[...] = jnp.maximum(v + b_ref[...], 0.0)


def _encode(x, w, b):
    return pl.pallas_call(
        _encode_body,
        grid=(NBLK,),
        in_specs=[pl.BlockSpec((NROW, NHID), lambda i: (i, 0)),
                  pl.BlockSpec((NHID, NHID), lambda i: (0, 0)),
                  pl.BlockSpec((1, NHID), lambda i: (0, 0))],
        out_specs=pl.BlockSpec((NROW, NHID), lambda i: (i, 0)),
        out_shape=jax.ShapeDtypeStruct((N, NHID), jnp.float32),
    )(x, w, b.reshape(1, NHID))


def _combine_body(h_ref, a_ref, w_ref, b_ref, o_ref):
    u = h_ref[...] + a_ref[0] + a_ref[1]
    v = jnp.dot(u, w_ref[...], preferred_element_type=jnp.float32) + b_ref[...]
    o_ref[...] = jnp.maximum(v, 0.0) + h_ref[...]


def _combine(h, agg, w, b):
    return pl.pallas_call(
        _combine_body,
        grid=(NBLK,),
        in_specs=[pl.BlockSpec((NROW, NHID), lambda i: (i, 0)),
                  pl.BlockSpec((NC, NROW, NHID), lambda i: (0, i, 0)),
                  pl.BlockSpec((NHID, NHID), lambda i: (0, 0)),
                  pl.BlockSpec((1, NHID), lambda i: (0, 0))],
        out_specs=pl.BlockSpec((NROW, NHID), lambda i: (i, 0)),
        out_shape=jax.ShapeDtypeStruct((N, NHID), jnp.float32),
    )(h, agg, w, b.reshape(1, NHID))


def _pool_body(h_ref, bt_ref, w1_ref, b1_ref, w2_ref, b2_ref, o_ref, acc_ref):
    i = pl.program_id(0)

    @pl.when(i == 0)
    def _():
        acc_ref[...] = jnp.zeros_like(acc_ref)

    row = bt_ref[0]
    oh = (lax.broadcasted_iota(jnp.int32, (NGRAPH, NROW), 0) == row)
    acc_ref[...] += jnp.dot(oh.astype(jnp.float32), h_ref[...],
                            preferred_element_type=jnp.float32)

    @pl.when(i == NBLK - 1)
    def _():
        p = jnp.dot(acc_ref[...], w1_ref[...],
                    preferred_element_type=jnp.float32) + b1_ref[...]
        p = jnp.maximum(p, 0.0)
        o_ref[...] = jnp.dot(p, w2_ref[...],
                             preferred_element_type=jnp.float32) + b2_ref[...]


def _pool(h, batch3, w1, b1, w2, b2):
    return pl.pallas_call(
        _pool_body,
        grid=(NBLK,),
        in_specs=[pl.BlockSpec((NROW, NHID), lambda i: (i, 0)),
                  pl.BlockSpec((1, 1, NROW), lambda i: (i, 0, 0)),
                  pl.BlockSpec((NHID, NHID), lambda i: (0, 0)),
                  pl.BlockSpec((1, NHID), lambda i: (0, 0)),
                  pl.BlockSpec((NHID, NOUT), lambda i: (0, 0)),
                  pl.BlockSpec((1, NOUT), lambda i: (0, 0))],
        out_specs=pl.BlockSpec((NGRAPH, NOUT), lambda i: (0, 0)),
        out_shape=jax.ShapeDtypeStruct((NGRAPH, NOUT), jnp.float32),
        scratch_shapes=[pltpu.VMEM((NGRAPH, NHID), jnp.float32)],
    )(h, batch3, w1, b1.reshape(1, NHID), w2, b2.reshape(1, NOUT))


def kernel(x, edge_attr, W_in, b_in, W_e, b_e, W_c, b_c, W_o1, b_o1, W_o2,
           b_o2, edge_index, batch):
    src = edge_index[0]
    dst = edge_index[1]
    ea2 = edge_attr.reshape(E // 2, 8)
    h = _encode(x, W_in, b_in)
    for l in range(3):
        e2 = _e_encode(ea2, W_e[l], b_e[l])
        agg = _edge_pass(h, e2.reshape(E * (NHID // 2)), src, dst)
        h = _combine(h, agg, W_c[l], b_c[l])
    return _pool(h, batch.reshape(NBLK, 1, NROW), W_o1, b_o1, W_o2, b_o2)


# speedup vs baseline: 26.6908x; 1.0031x over previous
import functools

import numpy as np
import jax
import jax.numpy as jnp
from jax import lax
from jax.experimental import pallas as pl
from jax.experimental.pallas import tpu as pltpu
from jax.experimental.pallas import tpu_sc as plsc

N = 10000
E = 320000
NHID = 128
NGRAPH = 64
NOUT = 64

NC, NS = 2, 16
NW = NC * NS
EPT = E // NW
CHUNK = 80
NCHUNK = EPT // CHUNK
NBUF = 2
RPS = 624
LAST_EXTRA = N - NS * RPS
ZR = 16
NROW = 1000
NBLK = N // NROW
EROW = 1000
EBLK = E // 2 // EROW


def _edge_body(h_hbm, e_hbm, src_hbm, dst_hbm, out_hbm,
               agg_sh, sbuf_b, dbuf_b, ebuf_b, rows_b, zed_v,
               sem_i, sem_d, sem_g, sem_s):
    c = lax.axis_index("c")
    s = lax.axis_index("s")
    wid = c * NS + s

    z = jnp.zeros((16,), jnp.float32)

    def zb(i, carry):
        zed_v[i // 8, pl.ds((i % 8) * 16, 16)] = z
        return carry

    lax.fori_loop(0, ZR * 8, zb, 0)
    for k in range(RPS // ZR):
        pltpu.sync_copy(zed_v, agg_sh.at[pl.ds(s * RPS + k * ZR, ZR)])

    @pl.when(s == NS - 1)
    def _():
        pltpu.sync_copy(zed_v, agg_sh.at[pl.ds(NS * RPS, LAST_EXTRA)])

    plsc.subcore_barrier()

    base0 = wid * EPT

    def _src(t, b):
        return pltpu.make_async_copy(
            src_hbm.at[pl.ds(base0 + t * CHUNK, CHUNK)], sbuf_b[b], sem_i[b])

    def _attr(t, b):
        return pltpu.make_async_copy(
            e_hbm.at[pl.ds(wid * (EPT // 2) + t * (CHUNK // 2), CHUNK // 2), :],
            ebuf_b[b], sem_i[b])

    def _dst(t, b):
        return pltpu.make_async_copy(
            dst_hbm.at[pl.ds(base0 + t * CHUNK, CHUNK)], dbuf_b[b], sem_d[b])

    def _gather(t, b):
        return pltpu.make_async_copy(
            h_hbm.at[sbuf_b[b]], rows_b[b], sem_g[b])

    def _scatter(t, b):
        return pltpu.make_async_copy(
            rows_b[b], agg_sh.at[dbuf_b[b]], sem_s[b])

    def _compute(t, b):
        rows_v = rows_b[b]
        ebuf = ebuf_b[b]

        def pair_body(p, icarry):
            for q in range(2):
                i = 2 * p + q
                for g in range(4):
                    iv = ebuf[p, pl.ds(q * 64 + g * 16, 16)]
                    ea = lax.bitcast_convert_type(iv << 16, jnp.float32)
                    eb = lax.bitcast_convert_type(iv & jnp.int32(-65536),
                                                  jnp.float32)
                    m0 = jnp.maximum(rows_v[i, pl.ds(g * 16, 16)] + ea, 0.0)
                    rows_v[i, pl.ds(g * 16, 16)] = m0
                    m1 = jnp.maximum(rows_v[i, pl.ds(64 + g * 16, 16)] + eb,
                                     0.0)
                    rows_v[i, pl.ds(64 + g * 16, 16)] = m1
            return icarry

        lax.fori_loop(0, CHUNK // 2, pair_body, 0)

    def step(t, b):
        @pl.when(t + 1 < NCHUNK)
        def _():
            @pl.when(t >= 1)
            def _():
                _scatter(t - 1, 1 - b).wait()

            _src(t + 1, 1 - b).wait()
            _attr(t + 1, 1 - b).wait()
            _gather(t + 1, 1 - b).start()
            _dst(t + 1, 1 - b).start()

        _gather(t, b).wait()
        _dst(t, b).wait()
        _compute(t, b)
        _scatter(t, b).start(add=True)

        @pl.when(t + 2 < NCHUNK)
        def _():
            _src(t + 2, b).start()
            _attr(t + 2, b).start()

    _src(0, 0).start()
    _attr(0, 0).start()
    _src(1, 1).start()
    _attr(1, 1).start()
    _dst(0, 0).start()
    _src(0, 0).wait()
    _attr(0, 0).wait()
    _gather(0, 0).start()

    def kiter(k, carry):
        step(2 * k, 0)
        step(2 * k + 1, 1)
        return carry

    lax.fori_loop(0, NCHUNK // 2, kiter, 0)
    step(NCHUNK - 1, (NCHUNK - 1) % 2)
    _scatter(NCHUNK - 2, (NCHUNK - 2) % 2).wait()
    _scatter(NCHUNK - 1, (NCHUNK - 1) % 2).wait()

    plsc.subcore_barrier()
    pltpu.sync_copy(agg_sh.at[pl.ds(s * RPS, RPS)],
                    out_hbm.at[c, pl.ds(s * RPS, RPS)])

    @pl.when(s == NS - 1)
    def _():
        pltpu.sync_copy(agg_sh.at[pl.ds(NS * RPS, LAST_EXTRA)],
                        out_hbm.at[c, pl.ds(NS * RPS, LAST_EXTRA)])


def _edge_pass(h, e_flat, src, dst):
    mesh = plsc.VectorSubcoreMesh(core_axis_name="c", subcore_axis_name="s",
                                  num_cores=NC, num_subcores=NS)
    f = pl.kernel(
        _edge_body,
        out_type=jax.ShapeDtypeStruct((NC, N, NHID), jnp.float32),
        mesh=mesh,
        scratch_types=[
            pltpu.VMEM_SHARED((N, NHID), jnp.float32),
            [pltpu.VMEM((CHUNK,), jnp.int32) for _ in range(NBUF)],
            [pltpu.VMEM((CHUNK,), jnp.int32) for _ in range(NBUF)],
            [pltpu.VMEM((CHUNK // 2, NHID), jnp.int32) for _ in range(NBUF)],
            [pltpu.VMEM((CHUNK, NHID), jnp.float32) for _ in range(NBUF)],
            pltpu.VMEM((ZR, NHID), jnp.float32),
            [pltpu.SemaphoreType.DMA for _ in range(NBUF)],
            [pltpu.SemaphoreType.DMA for _ in range(NBUF)],
            [pltpu.SemaphoreType.DMA for _ in range(NBUF)],
            [pltpu.SemaphoreType.DMA for _ in range(NBUF)],
        ],
    )
    return f(h, e_flat, src, dst)


def _e_encode_body(a_ref, w_ref, b_ref, o_ref):
    halves = []
    for q in range(2):
        a = a_ref[:, 4 * q:4 * q + 4]
        v = jnp.dot(a, w_ref[...], preferred_element_type=jnp.float32)
        v = jnp.maximum(v + b_ref[...], 0.0)
        bits = lax.bitcast_convert_type(v, jnp.int32)
        rnd = (bits + 0x7FFF + ((bits >> 16) & 1)) >> 16
        halves.append((rnd[:, :64] & 0xFFFF) | (rnd[:, 64:] << 16))
    o_ref[...] = jnp.concatenate(halves, axis=1)


def _e_encode(edge_attr2, w, b):
    return pl.pallas_call(
        _e_encode_body,
        grid=(EBLK,),
        in_specs=[pl.BlockSpec((EROW, 8), lambda i: (i, 0)),
                  pl.BlockSpec((4, NHID), lambda i: (0, 0)),
                  pl.BlockSpec((1, NHID), lambda i: (0, 0))],
        out_specs=pl.BlockSpec((EROW, NHID), lambda i: (i, 0)),
        out_shape=jax.ShapeDtypeStruct((E // 2, NHID), jnp.int32),
    )(edge_attr2, w, b.reshape(1, NHID))


def _encode_body(x_ref, w_ref, b_ref, o_ref):
    v = jnp.dot(x_ref[...], w_ref[...], preferred_element_type=jnp.float32)
    o_ref[...] = jnp.maximum(v + b_ref[...], 0.0)


def _encode(x, w, b):
    return pl.pallas_call(
        _encode_body,
        grid=(NBLK,),
        in_specs=[pl.BlockSpec((NROW, NHID), lambda i: (i, 0)),
                  pl.BlockSpec((NHID, NHID), lambda i: (0, 0)),
                  pl.BlockSpec((1, NHID), lambda i: (0, 0))],
        out_specs=pl.BlockSpec((NROW, NHID), lambda i: (i, 0)),
        out_shape=jax.ShapeDtypeStruct((N, NHID), jnp.float32),
    )(x, w, b.reshape(1, NHID))


def _combine_body(h_ref, a_ref, w_ref, b_ref, o_ref):
    u = h_ref[...] + a_ref[0] + a_ref[1]
    v = jnp.dot(u, w_ref[...], preferred_element_type=jnp.float32) + b_ref[...]
    o_ref[...] = jnp.maximum(v, 0.0) + h_ref[...]


def _combine(h, agg, w, b):
    return pl.pallas_call(
        _combine_body,
        grid=(NBLK,),
        in_specs=[pl.BlockSpec((NROW, NHID), lambda i: (i, 0)),
                  pl.BlockSpec((NC, NROW, NHID), lambda i: (0, i, 0)),
                  pl.BlockSpec((NHID, NHID), lambda i: (0, 0)),
                  pl.BlockSpec((1, NHID), lambda i: (0, 0))],
        out_specs=pl.BlockSpec((NROW, NHID), lambda i: (i, 0)),
        out_shape=jax.ShapeDtypeStruct((N, NHID), jnp.float32),
    )(h, agg, w, b.reshape(1, NHID))


def _pool_body(h_ref, bt_ref, w1_ref, b1_ref, w2_ref, b2_ref, o_ref, acc_ref):
    i = pl.program_id(0)

    @pl.when(i == 0)
    def _():
        acc_ref[...] = jnp.zeros_like(acc_ref)

    row = bt_ref[0]
    oh = (lax.broadcasted_iota(jnp.int32, (NGRAPH, NROW), 0) == row)
    acc_ref[...] += jnp.dot(oh.astype(jnp.float32), h_ref[...],
                            preferred_element_type=jnp.float32)

    @pl.when(i == NBLK - 1)
    def _():
        p = jnp.dot(acc_ref[...], w1_ref[...],
                    preferred_element_type=jnp.float32) + b1_ref[...]
        p = jnp.maximum(p, 0.0)
        o_ref[...] = jnp.dot(p, w2_ref[...],
                             preferred_element_type=jnp.float32) + b2_ref[...]


def _pool(h, batch3, w1, b1, w2, b2):
    return pl.pallas_call(
        _pool_body,
        grid=(NBLK,),
        in_specs=[pl.BlockSpec((NROW, NHID), lambda i: (i, 0)),
                  pl.BlockSpec((1, 1, NROW), lambda i: (i, 0, 0)),
                  pl.BlockSpec((NHID, NHID), lambda i: (0, 0)),
                  pl.BlockSpec((1, NHID), lambda i: (0, 0)),
                  pl.BlockSpec((NHID, NOUT), lambda i: (0, 0)),
                  pl.BlockSpec((1, NOUT), lambda i: (0, 0))],
        out_specs=pl.BlockSpec((NGRAPH, NOUT), lambda i: (0, 0)),
        out_shape=jax.ShapeDtypeStruct((NGRAPH, NOUT), jnp.float32),
        scratch_shapes=[pltpu.VMEM((NGRAPH, NHID), jnp.float32)],
    )(h, batch3, w1, b1.reshape(1, NHID), w2, b2.reshape(1, NOUT))


def kernel(x, edge_attr, W_in, b_in, W_e, b_e, W_c, b_c, W_o1, b_o1, W_o2,
           b_o2, edge_index, batch):
    src = edge_index[0]
    dst = edge_index[1]
    ea2 = edge_attr.reshape(E // 2, 8)
    h = _encode(x, W_in, b_in)
    for l in range(3):
        e2 = _e_encode(ea2, W_e[l], b_e[l])
        agg = _edge_pass(h, e2, src, dst)
        h = _combine(h, agg, W_c[l], b_c[l])
    return _pool(h, batch.reshape(NBLK, 1, NROW), W_o1, b_o1, W_o2, b_o2)


# speedup vs baseline: 36.9759x; 1.3853x over previous
import functools

import numpy as np
import jax
import jax.numpy as jnp
from jax import lax
from jax.experimental import pallas as pl
from jax.experimental.pallas import tpu as pltpu
from jax.experimental.pallas import tpu_sc as plsc

N = 10000
E = 320000
NHID = 128
NGRAPH = 64
NOUT = 64

NC, NS = 2, 16
NW = NC * NS
EPT = E // NW
CHUNK = 80
NCHUNK = EPT // CHUNK
NBUF = 2
RPS = 624
LAST_EXTRA = N - NS * RPS
ZR = 16
NROW = 1000
NBLK = N // NROW


def _edge_body(h_hbm, attr_hbm, we_hbm, be_hbm, src_hbm, dst_hbm, out_hbm,
               agg_sh, sbuf_b, dbuf_b, abuf_b, rows_b, we_v, be_v, zed_v,
               sem_i, sem_d, sem_g, sem_s):
    c = lax.axis_index("c")
    s = lax.axis_index("s")
    wid = c * NS + s

    z = jnp.zeros((16,), jnp.float32)

    def zb(i, carry):
        zed_v[i // 8, pl.ds((i % 8) * 16, 16)] = z
        return carry

    lax.fori_loop(0, ZR * 8, zb, 0)
    for k in range(RPS // ZR):
        pltpu.sync_copy(zed_v, agg_sh.at[pl.ds(s * RPS + k * ZR, ZR)])

    @pl.when(s == NS - 1)
    def _():
        pltpu.sync_copy(zed_v, agg_sh.at[pl.ds(NS * RPS, LAST_EXTRA)])

    pltpu.sync_copy(we_hbm, we_v)
    pltpu.sync_copy(be_hbm, be_v)
    wvec = [[we_v[k, pl.ds(r * 16, 16)] for r in range(8)] for k in range(4)]
    bvec = [be_v[pl.ds(r * 16, 16)] for r in range(8)]

    plsc.subcore_barrier()

    base0 = wid * EPT

    def _src(t, b):
        return pltpu.make_async_copy(
            src_hbm.at[pl.ds(base0 + t * CHUNK, CHUNK)], sbuf_b[b], sem_i[b])

    def _attr(t, b):
        return pltpu.make_async_copy(
            attr_hbm.at[pl.ds((base0 + t * CHUNK) * 4, CHUNK * 4)],
            abuf_b[b], sem_i[b])

    def _dst(t, b):
        return pltpu.make_async_copy(
            dst_hbm.at[pl.ds(base0 + t * CHUNK, CHUNK)], dbuf_b[b], sem_d[b])

    def _gather(t, b):
        return pltpu.make_async_copy(
            h_hbm.at[sbuf_b[b]], rows_b[b], sem_g[b])

    def _scatter(t, b):
        return pltpu.make_async_copy(
            rows_b[b], agg_sh.at[dbuf_b[b]], sem_s[b])

    def _compute(t, b):
        rows_v = rows_b[b]
        abuf = abuf_b[b]

        def group_body(g, icarry):
            av = abuf[pl.ds(g * 16, 16)]
            for q in range(4):
                i = g * 4 + q
                for r in range(8):
                    e = (av[4 * q] * wvec[0][r] + av[4 * q + 1] * wvec[1][r]
                         + av[4 * q + 2] * wvec[2][r]
                         + av[4 * q + 3] * wvec[3][r] + bvec[r])
                    e = jnp.maximum(e, 0.0)
                    m = jnp.maximum(rows_v[i, pl.ds(r * 16, 16)] + e, 0.0)
                    rows_v[i, pl.ds(r * 16, 16)] = m
            return icarry

        lax.fori_loop(0, CHUNK // 4, group_body, 0)

    def step(t, b):
        @pl.when(t + 1 < NCHUNK)
        def _():
            @pl.when(t >= 1)
            def _():
                _scatter(t - 1, 1 - b).wait()

            _src(t + 1, 1 - b).wait()
            _attr(t + 1, 1 - b).wait()
            _gather(t + 1, 1 - b).start()
            _dst(t + 1, 1 - b).start()

        _gather(t, b).wait()
        _dst(t, b).wait()
        _compute(t, b)
        _scatter(t, b).start(add=True)

        @pl.when(t + 2 < NCHUNK)
        def _():
            _src(t + 2, b).start()
            _attr(t + 2, b).start()

    _src(0, 0).start()
    _attr(0, 0).start()
    _src(1, 1).start()
    _attr(1, 1).start()
    _dst(0, 0).start()
    _src(0, 0).wait()
    _attr(0, 0).wait()
    _gather(0, 0).start()

    def kiter(k, carry):
        step(2 * k, 0)
        step(2 * k + 1, 1)
        return carry

    lax.fori_loop(0, NCHUNK // 2, kiter, 0)
    step(NCHUNK - 1, (NCHUNK - 1) % 2)
    _scatter(NCHUNK - 2, (NCHUNK - 2) % 2).wait()
    _scatter(NCHUNK - 1, (NCHUNK - 1) % 2).wait()

    plsc.subcore_barrier()
    pltpu.sync_copy(agg_sh.at[pl.ds(s * RPS, RPS)],
                    out_hbm.at[c, pl.ds(s * RPS, RPS)])

    @pl.when(s == NS - 1)
    def _():
        pltpu.sync_copy(agg_sh.at[pl.ds(NS * RPS, LAST_EXTRA)],
                        out_hbm.at[c, pl.ds(NS * RPS, LAST_EXTRA)])


def _edge_pass(h, edge_attr, we, be, src, dst):
    edge_attr = edge_attr.reshape(E * 4)
    mesh = plsc.VectorSubcoreMesh(core_axis_name="c", subcore_axis_name="s",
                                  num_cores=NC, num_subcores=NS)
    f = pl.kernel(
        _edge_body,
        out_type=jax.ShapeDtypeStruct((NC, N, NHID), jnp.float32),
        mesh=mesh,
        scratch_types=[
            pltpu.VMEM_SHARED((N, NHID), jnp.float32),
            [pltpu.VMEM((CHUNK,), jnp.int32) for _ in range(NBUF)],
            [pltpu.VMEM((CHUNK,), jnp.int32) for _ in range(NBUF)],
            [pltpu.VMEM((CHUNK * 4,), jnp.float32) for _ in range(NBUF)],
            [pltpu.VMEM((CHUNK, NHID), jnp.float32) for _ in range(NBUF)],
            pltpu.VMEM((4, NHID), jnp.float32),
            pltpu.VMEM((NHID,), jnp.float32),
            pltpu.VMEM((ZR, NHID), jnp.float32),
            [pltpu.SemaphoreType.DMA for _ in range(NBUF)],
            [pltpu.SemaphoreType.DMA for _ in range(NBUF)],
            [pltpu.SemaphoreType.DMA for _ in range(NBUF)],
            [pltpu.SemaphoreType.DMA for _ in range(NBUF)],
        ],
    )
    return f(h, edge_attr, we, be, src, dst)


def _encode_body(x_ref, w_ref, b_ref, o_ref):
    v = jnp.dot(x_ref[...], w_ref[...], preferred_element_type=jnp.float32)
    o_ref[...] = jnp.maximum(v + b_ref[...], 0.0)


def _encode(x, w, b):
    return pl.pallas_call(
        _encode_body,
        grid=(NBLK,),
        in_specs=[pl.BlockSpec((NROW, NHID), lambda i: (i, 0)),
                  pl.BlockSpec((NHID, NHID), lambda i: (0, 0)),
                  pl.BlockSpec((1, NHID), lambda i: (0, 0))],
        out_specs=pl.BlockSpec((NROW, NHID), lambda i: (i, 0)),
        out_shape=jax.ShapeDtypeStruct((N, NHID), jnp.float32),
    )(x, w, b.reshape(1, NHID))


def _combine_body(h_ref, a_ref, w_ref, b_ref, o_ref):
    u = h_ref[...] + a_ref[0] + a_ref[1]
    v = jnp.dot(u, w_ref[...], preferred_element_type=jnp.float32) + b_ref[...]
    o_ref[...] = jnp.maximum(v, 0.0) + h_ref[...]


def _combine(h, agg, w, b):
    return pl.pallas_call(
        _combine_body,
        grid=(NBLK,),
        in_specs=[pl.BlockSpec((NROW, NHID), lambda i: (i, 0)),
                  pl.BlockSpec((NC, NROW, NHID), lambda i: (0, i, 0)),
                  pl.BlockSpec((NHID, NHID), lambda i: (0, 0)),
                  pl.BlockSpec((1, NHID), lambda i: (0, 0))],
        out_specs=pl.BlockSpec((NROW, NHID), lambda i: (i, 0)),
        out_shape=jax.ShapeDtypeStruct((N, NHID), jnp.float32),
    )(h, agg, w, b.reshape(1, NHID))


def _pool_body(h_ref, bt_ref, w1_ref, b1_ref, w2_ref, b2_ref, o_ref, acc_ref):
    i = pl.program_id(0)

    @pl.when(i == 0)
    def _():
        acc_ref[...] = jnp.zeros_like(acc_ref)

    row = bt_ref[0]
    oh = (lax.broadcasted_iota(jnp.int32, (NGRAPH, NROW), 0) == row)
    acc_ref[...] += jnp.dot(oh.astype(jnp.float32), h_ref[...],
                            preferred_element_type=jnp.float32)

    @pl.when(i == NBLK - 1)
    def _():
        p = jnp.dot(acc_ref[...], w1_ref[...],
                    preferred_element_type=jnp.float32) + b1_ref[...]
        p = jnp.maximum(p, 0.0)
        o_ref[...] = jnp.dot(p, w2_ref[...],
                             preferred_element_type=jnp.float32) + b2_ref[...]


def _pool(h, batch3, w1, b1, w2, b2):
    return pl.pallas_call(
        _pool_body,
        grid=(NBLK,),
        in_specs=[pl.BlockSpec((NROW, NHID), lambda i: (i, 0)),
                  pl.BlockSpec((1, 1, NROW), lambda i: (i, 0, 0)),
                  pl.BlockSpec((NHID, NHID), lambda i: (0, 0)),
                  pl.BlockSpec((1, NHID), lambda i: (0, 0)),
                  pl.BlockSpec((NHID, NOUT), lambda i: (0, 0)),
                  pl.BlockSpec((1, NOUT), lambda i: (0, 0))],
        out_specs=pl.BlockSpec((NGRAPH, NOUT), lambda i: (0, 0)),
        out_shape=jax.ShapeDtypeStruct((NGRAPH, NOUT), jnp.float32),
        scratch_shapes=[pltpu.VMEM((NGRAPH, NHID), jnp.float32)],
    )(h, batch3, w1, b1.reshape(1, NHID), w2, b2.reshape(1, NOUT))


def kernel(x, edge_attr, W_in, b_in, W_e, b_e, W_c, b_c, W_o1, b_o1, W_o2,
           b_o2, edge_index, batch):
    src = edge_index[0]
    dst = edge_index[1]
    h = _encode(x, W_in, b_in)
    for l in range(3):
        agg = _edge_pass(h, edge_attr, W_e[l], b_e[l], src, dst)
        h = _combine(h, agg, W_c[l], b_c[l])
    return _pool(h, batch.reshape(NBLK, 1, NROW), W_o1, b_o1, W_o2, b_o2)


# speedup vs baseline: 36.9931x; 1.0005x over previous
import jax
import jax.numpy as jnp
from jax import lax
from jax.experimental import pallas as pl
from jax.experimental.pallas import tpu as pltpu
from jax.experimental.pallas import tpu_sc as plsc

N = 10000
E = 320000
NHID = 128
NGRAPH = 64
NOUT = 64

NC, NS = 2, 16
NW = NC * NS
EPT = E // NW
CHUNK = 80
NCHUNK = EPT // CHUNK
NBUF = 2
RPS = 624
LAST_EXTRA = N - NS * RPS
ZR = 16
NROW = 1000
NBLK = N // NROW


def _edge_body(h_hbm, attr_hbm, we_hbm, be_hbm, src_hbm, dst_hbm, out_hbm,
               agg_sh, sbuf_b, dbuf_b, abuf_b, rows_b, we_v, be_v, zed_v,
               sem_i, sem_d, sem_g, sem_s):
    c = lax.axis_index("c")
    s = lax.axis_index("s")
    wid = c * NS + s

    z = jnp.zeros((16,), jnp.float32)

    def zb(i, carry):
        zed_v[i // 8, pl.ds((i % 8) * 16, 16)] = z
        return carry

    lax.fori_loop(0, ZR * 8, zb, 0)
    for k in range(RPS // ZR):
        pltpu.sync_copy(zed_v, agg_sh.at[pl.ds(s * RPS + k * ZR, ZR)])

    @pl.when(s == NS - 1)
    def _():
        pltpu.sync_copy(zed_v, agg_sh.at[pl.ds(NS * RPS, LAST_EXTRA)])

    pltpu.sync_copy(we_hbm, we_v)
    pltpu.sync_copy(be_hbm, be_v)
    wvec = [[we_v[k, pl.ds(r * 16, 16)] for r in range(8)] for k in range(4)]
    bvec = [be_v[pl.ds(r * 16, 16)] for r in range(8)]

    plsc.subcore_barrier()

    base0 = wid * EPT

    def _src(t, b):
        return pltpu.make_async_copy(
            src_hbm.at[pl.ds(base0 + t * CHUNK, CHUNK)], sbuf_b[b], sem_i[b])

    def _attr(t, b):
        return pltpu.make_async_copy(
            attr_hbm.at[pl.ds((base0 + t * CHUNK) * 4, CHUNK * 4)],
            abuf_b[b], sem_i[b])

    def _dst(t, b):
        return pltpu.make_async_copy(
            dst_hbm.at[pl.ds(base0 + t * CHUNK, CHUNK)], dbuf_b[b], sem_d[b])

    def _gather(t, b):
        return pltpu.make_async_copy(
            h_hbm.at[sbuf_b[b]], rows_b[b], sem_g[b])

    def _scatter(t, b):
        return pltpu.make_async_copy(
            rows_b[b], agg_sh.at[dbuf_b[b]], sem_s[b])

    def _compute(t, b):
        rows_v = rows_b[b]
        abuf = abuf_b[b]

        def group_body(g, icarry):
            av = abuf[pl.ds(g * 16, 16)]
            for q in range(4):
                i = g * 4 + q
                for r in range(8):
                    e = (av[4 * q] * wvec[0][r] + av[4 * q + 1] * wvec[1][r]
                         + av[4 * q + 2] * wvec[2][r]
                         + av[4 * q + 3] * wvec[3][r] + bvec[r])
                    e = jnp.maximum(e, 0.0)
                    m = jnp.maximum(rows_v[i, pl.ds(r * 16, 16)] + e, 0.0)
                    rows_v[i, pl.ds(r * 16, 16)] = m
            return icarry

        lax.fori_loop(0, CHUNK // 4, group_body, 0)

    def step(t, b):
        @pl.when(t + 1 < NCHUNK)
        def _():
            @pl.when(t >= 1)
            def _():
                _scatter(t - 1, 1 - b).wait()

            _src(t + 1, 1 - b).wait()
            _attr(t + 1, 1 - b).wait()
            _gather(t + 1, 1 - b).start()
            _dst(t + 1, 1 - b).start()

        _gather(t, b).wait()
        _dst(t, b).wait()
        _compute(t, b)
        _scatter(t, b).start(add=True)

        @pl.when(t + 2 < NCHUNK)
        def _():
            _src(t + 2, b).start()
            _attr(t + 2, b).start()

    _src(0, 0).start()
    _attr(0, 0).start()
    _src(1, 1).start()
    _attr(1, 1).start()
    _dst(0, 0).start()
    _src(0, 0).wait()
    _attr(0, 0).wait()
    _gather(0, 0).start()

    def kiter(k, carry):
        step(2 * k, 0)
        step(2 * k + 1, 1)
        return carry

    lax.fori_loop(0, NCHUNK // 2, kiter, 0)
    step(NCHUNK - 1, (NCHUNK - 1) % 2)
    _scatter(NCHUNK - 2, (NCHUNK - 2) % 2).wait()
    _scatter(NCHUNK - 1, (NCHUNK - 1) % 2).wait()

    plsc.subcore_barrier()
    pltpu.sync_copy(agg_sh.at[pl.ds(s * RPS, RPS)],
                    out_hbm.at[c, pl.ds(s * RPS, RPS)])

    @pl.when(s == NS - 1)
    def _():
        pltpu.sync_copy(agg_sh.at[pl.ds(NS * RPS, LAST_EXTRA)],
                        out_hbm.at[c, pl.ds(NS * RPS, LAST_EXTRA)])


def _edge_pass(h, edge_attr, we, be, src, dst):
    edge_attr = edge_attr.reshape(E * 4)
    mesh = plsc.VectorSubcoreMesh(core_axis_name="c", subcore_axis_name="s",
                                  num_cores=NC, num_subcores=NS)
    f = pl.kernel(
        _edge_body,
        out_type=jax.ShapeDtypeStruct((NC, N, NHID), jnp.float32),
        mesh=mesh,
        scratch_types=[
            pltpu.VMEM_SHARED((N, NHID), jnp.float32),
            [pltpu.VMEM((CHUNK,), jnp.int32) for _ in range(NBUF)],
            [pltpu.VMEM((CHUNK,), jnp.int32) for _ in range(NBUF)],
            [pltpu.VMEM((CHUNK * 4,), jnp.float32) for _ in range(NBUF)],
            [pltpu.VMEM((CHUNK, NHID), jnp.float32) for _ in range(NBUF)],
            pltpu.VMEM((4, NHID), jnp.float32),
            pltpu.VMEM((NHID,), jnp.float32),
            pltpu.VMEM((ZR, NHID), jnp.float32),
            [pltpu.SemaphoreType.DMA for _ in range(NBUF)],
            [pltpu.SemaphoreType.DMA for _ in range(NBUF)],
            [pltpu.SemaphoreType.DMA for _ in range(NBUF)],
            [pltpu.SemaphoreType.DMA for _ in range(NBUF)],
        ],
    )
    return f(h, edge_attr, we, be, src, dst)


def _encode_body(x_ref, w_ref, b_ref, o_ref):
    v = jnp.dot(x_ref[...], w_ref[...], preferred_element_type=jnp.float32)
    o_ref[...] = jnp.maximum(v + b_ref[...], 0.0)


def _encode(x, w, b):
    return pl.pallas_call(
        _encode_body,
        grid=(NBLK,),
        in_specs=[pl.BlockSpec((NROW, NHID), lambda i: (i, 0)),
                  pl.BlockSpec((NHID, NHID), lambda i: (0, 0)),
                  pl.BlockSpec((1, NHID), lambda i: (0, 0))],
        out_specs=pl.BlockSpec((NROW, NHID), lambda i: (i, 0)),
        out_shape=jax.ShapeDtypeStruct((N, NHID), jnp.float32),
    )(x, w, b.reshape(1, NHID))


def _combine_body(h_ref, a_ref, w_ref, b_ref, o_ref):
    u = h_ref[...] + a_ref[0] + a_ref[1]
    v = jnp.dot(u, w_ref[...], preferred_element_type=jnp.float32) + b_ref[...]
    o_ref[...] = jnp.maximum(v, 0.0) + h_ref[...]


def _combine(h, agg, w, b):
    return pl.pallas_call(
        _combine_body,
        grid=(NBLK,),
        in_specs=[pl.BlockSpec((NROW, NHID), lambda i: (i, 0)),
                  pl.BlockSpec((NC, NROW, NHID), lambda i: (0, i, 0)),
                  pl.BlockSpec((NHID, NHID), lambda i: (0, 0)),
                  pl.BlockSpec((1, NHID), lambda i: (0, 0))],
        out_specs=pl.BlockSpec((NROW, NHID), lambda i: (i, 0)),
        out_shape=jax.ShapeDtypeStruct((N, NHID), jnp.float32),
    )(h, agg, w, b.reshape(1, NHID))


def _pool_body(h_ref, bt_ref, w1_ref, b1_ref, w2_ref, b2_ref, o_ref, acc_ref):
    i = pl.program_id(0)

    @pl.when(i == 0)
    def _():
        acc_ref[...] = jnp.zeros_like(acc_ref)

    row = bt_ref[0]
    oh = (lax.broadcasted_iota(jnp.int32, (NGRAPH, NROW), 0) == row)
    acc_ref[...] += jnp.dot(oh.astype(jnp.float32), h_ref[...],
                            preferred_element_type=jnp.float32)

    @pl.when(i == NBLK - 1)
    def _():
        p = jnp.dot(acc_ref[...], w1_ref[...],
                    preferred_element_type=jnp.float32) + b1_ref[...]
        p = jnp.maximum(p, 0.0)
        o_ref[...] = jnp.dot(p, w2_ref[...],
                             preferred_element_type=jnp.float32) + b2_ref[...]


def _pool(h, batch3, w1, b1, w2, b2):
    return pl.pallas_call(
        _pool_body,
        grid=(NBLK,),
        in_specs=[pl.BlockSpec((NROW, NHID), lambda i: (i, 0)),
                  pl.BlockSpec((1, 1, NROW), lambda i: (i, 0, 0)),
                  pl.BlockSpec((NHID, NHID), lambda i: (0, 0)),
                  pl.BlockSpec((1, NHID), lambda i: (0, 0)),
                  pl.BlockSpec((NHID, NOUT), lambda i: (0, 0)),
                  pl.BlockSpec((1, NOUT), lambda i: (0, 0))],
        out_specs=pl.BlockSpec((NGRAPH, NOUT), lambda i: (0, 0)),
        out_shape=jax.ShapeDtypeStruct((NGRAPH, NOUT), jnp.float32),
        scratch_shapes=[pltpu.VMEM((NGRAPH, NHID), jnp.float32)],
    )(h, batch3, w1, b1.reshape(1, NHID), w2, b2.reshape(1, NOUT))


def kernel(x, edge_attr, W_in, b_in, W_e, b_e, W_c, b_c, W_o1, b_o1, W_o2,
           b_o2, edge_index, batch):
    src = edge_index[0]
    dst = edge_index[1]
    h = _encode(x, W_in, b_in)
    for l in range(3):
        agg = _edge_pass(h, edge_attr, W_e[l], b_e[l], src, dst)
        h = _combine(h, agg, W_c[l], b_c[l])
    return _pool(h, batch.reshape(NBLK, 1, NROW), W_o1, b_o1, W_o2, b_o2)
